# baseline (speedup 1.0000x reference)
import functools

import numpy as np

import jax
import jax.numpy as jnp
from jax.experimental import pallas as pl
from jax.sharding import Mesh, PartitionSpec as P

try:
    from jax.experimental.shard_map import shard_map as _shard_map
except ImportError:
    from jax import shard_map as _shard_map


def _conv_stats_kernel(x_ref, w_ref, h_ref, s_ref, *, rb, n_s, co):
    b = pl.program_id(0)
    r = pl.program_id(1)

    @pl.when((b == 0) & (r == 0))
    def _init():
        s_ref[...] = jnp.zeros_like(s_ref)

    acc = jnp.zeros((rb * n_s, co), jnp.float32)
    for ky in range(3):
        base = (r * rb + ky) * n_s
        for kx in range(3):
            xs = x_ref[0, kx, pl.ds(base, rb * n_s), :]
            acc += jnp.dot(xs, w_ref[ky * 3 + kx],
                           preferred_element_type=jnp.float32)
    s_ref[...] += jnp.concatenate(
        [jnp.sum(acc, axis=0, keepdims=True),
         jnp.sum(acc * acc, axis=0, keepdims=True)], axis=0)
    h_ref[0] = acc.astype(jnp.bfloat16)


def _bn_proj_kernel(h_ref, s_ref, gb_ref, w2_ref, b2_ref, o_ref, *,
                    n_total, eps):
    inv_n = 1.0 / n_total
    mean = s_ref[0:1, :] * inv_n
    var = s_ref[1:2, :] * inv_n - mean * mean
    rstd = jax.lax.rsqrt(var + eps)
    scale = gb_ref[0:1, :] * rstd
    shift = gb_ref[1:2, :] - mean * scale
    y = h_ref[...].astype(jnp.float32) * scale + shift
    y = jnp.where(y > 0, y, 0.1 * y).astype(jnp.bfloat16)
    o_ref[...] = (jnp.dot(y, w2_ref[...], preferred_element_type=jnp.float32)
                  + b2_ref[...])


def _pass1(x, p, rb):
    B, ic, S, _ = x.shape
    co = p['w1'].shape[0]

    xp = jnp.pad(jnp.transpose(x, (0, 2, 3, 1)),
                 ((0, 0), (1, 1), (1, 1), (0, 0))).astype(jnp.bfloat16)
    xf = jnp.stack([xp[:, :, k:k + S, :].reshape(B, (S + 2) * S, ic)
                    for k in range(3)], axis=1)
    w1t = jnp.transpose(p['w1'], (2, 3, 1, 0)).reshape(9, ic, co)
    w1t = w1t.astype(jnp.bfloat16)

    nrb = S // rb
    return pl.pallas_call(
        functools.partial(_conv_stats_kernel, rb=rb, n_s=S, co=co),
        grid=(B, nrb),
        in_specs=[
            pl.BlockSpec((1, 3, (S + 2) * S, ic), lambda b, r: (b, 0, 0, 0)),
            pl.BlockSpec((9, ic, co), lambda b, r: (0, 0, 0)),
        ],
        out_specs=[
            pl.BlockSpec((1, rb * S, co), lambda b, r: (b, r, 0)),
            pl.BlockSpec((2, co), lambda b, r: (0, 0)),
        ],
        out_shape=[
            jax.ShapeDtypeStruct((B, S * S, co), jnp.bfloat16),
            jax.ShapeDtypeStruct((2, co), jnp.float32),
        ],
    )(xf, w1t)


def _pass2(h1, stats, p, n_total, mb):
    B, _, co = h1.shape
    S = int(round((h1.shape[1]) ** 0.5))
    no = p['w2'].shape[0]
    nop = ((no + 127) // 128) * 128

    M = B * S * S
    h1f = h1.reshape(M, co)
    gb = jnp.stack([p['g'], p['b']], axis=0).astype(jnp.float32)
    w2t = jnp.pad(p['w2'].reshape(no, co).T, ((0, 0), (0, nop - no)))
    w2t = w2t.astype(jnp.bfloat16)
    b2p = jnp.pad(p['b2'], (0, nop - no)).reshape(1, nop).astype(jnp.float32)

    out = pl.pallas_call(
        functools.partial(_bn_proj_kernel, n_total=float(n_total), eps=1e-5),
        grid=(M // mb,),
        in_specs=[
            pl.BlockSpec((mb, co), lambda i: (i, 0)),
            pl.BlockSpec((2, co), lambda i: (0, 0)),
            pl.BlockSpec((2, co), lambda i: (0, 0)),
            pl.BlockSpec((co, nop), lambda i: (0, 0)),
            pl.BlockSpec((1, nop), lambda i: (0, 0)),
        ],
        out_specs=pl.BlockSpec((mb, nop), lambda i: (i, 0)),
        out_shape=jax.ShapeDtypeStruct((M, nop), jnp.float32),
    )(h1f, stats, gb, w2t, b2p)
    return out.reshape(B, S, S, nop)[..., :no]


_SCALE_CFG = ((32, 2048), (32, 2048), (16, 512))


def _heads_local(feat0, feat1, feat2, params, b_global, axis):
    feats = (feat0, feat1, feat2)
    p1 = [_pass1(x, p, rb)
          for x, p, (rb, _) in zip(feats, params, _SCALE_CFG)]
    if axis is not None:
        allstats = jax.lax.psum(
            jnp.concatenate([s for _, s in p1], axis=1), axis)
        stats = jnp.split(allstats, 3, axis=1)
    else:
        stats = [s for _, s in p1]
    outs = []
    for (h1, _), st, x, p, (_, mb) in zip(p1, stats, feats, params,
                                          _SCALE_CFG):
        S = x.shape[2]
        m_loc = h1.shape[0] * S * S
        outs.append(_pass2(h1, st, p, n_total=b_global * S * S,
                           mb=min(mb, m_loc)))
    return tuple(outs)


def kernel(feat0, feat1, feat2, params):
    devs = jax.devices()
    B = feat0.shape[0]
    nd = 2 if (len(devs) >= 2 and B % 2 == 0) else 1
    if nd == 1:
        return _heads_local(feat0, feat1, feat2, params, B, None)
    mesh = Mesh(np.array(devs[:nd]), ('d',))
    f = _shard_map(
        lambda f0, f1, f2, ps: _heads_local(f0, f1, f2, ps, B, 'd'),
        mesh=mesh,
        in_specs=(P('d'), P('d'), P('d'), P()),
        out_specs=(P('d'), P('d'), P('d')),
        check_rep=False,
    )
    return f(feat0, feat1, feat2, params)


# speedup vs baseline: 1.3807x; 1.3807x over previous
import functools

import jax
import jax.numpy as jnp
from jax.experimental import pallas as pl


def _conv_stats_kernel(x_ref, w_ref, h_ref, s_ref, *, rb, n_s, co):
    b = pl.program_id(0)
    r = pl.program_id(1)

    @pl.when((b == 0) & (r == 0))
    def _init():
        s_ref[...] = jnp.zeros_like(s_ref)

    acc = jnp.zeros((rb * n_s, co), jnp.float32)
    for ky in range(3):
        base = (r * rb + ky) * n_s
        acc += jnp.dot(x_ref[0, pl.ds(base, rb * n_s), :], w_ref[ky],
                       preferred_element_type=jnp.float32)
    s_ref[...] += jnp.concatenate(
        [jnp.sum(acc, axis=0, keepdims=True),
         jnp.sum(acc * acc, axis=0, keepdims=True)], axis=0)
    h_ref[0] = acc.astype(jnp.bfloat16)


def _bn_proj_kernel(h_ref, s_ref, gb_ref, w2_ref, b2_ref, o_ref, *,
                    n_total, eps):
    inv_n = 1.0 / n_total
    mean = s_ref[0:1, :] * inv_n
    var = s_ref[1:2, :] * inv_n - mean * mean
    rstd = jax.lax.rsqrt(var + eps)
    scale = (gb_ref[0:1, :] * rstd).astype(jnp.bfloat16)
    shift = (gb_ref[1:2, :] - mean * gb_ref[0:1, :] * rstd)
    shift = shift.astype(jnp.bfloat16)
    z = h_ref[...] * scale + shift
    y = jnp.maximum(z, jnp.bfloat16(0.1) * z)
    o_ref[...] = (jnp.dot(y, w2_ref[...], preferred_element_type=jnp.float32)
                  + b2_ref[...])


def _pass1(x, p, rb):
    B, ic, S, _ = x.shape
    co = p['w1'].shape[0]

    xp = jnp.pad(jnp.transpose(x, (0, 2, 3, 1)),
                 ((0, 0), (1, 1), (1, 1), (0, 0))).astype(jnp.bfloat16)
    xf = jnp.concatenate([xp[:, :, k:k + S, :] for k in range(3)],
                         axis=3).reshape(B, (S + 2) * S, 3 * ic)
    w1t = jnp.transpose(p['w1'], (2, 3, 1, 0)).reshape(3, 3 * ic, co)
    w1t = w1t.astype(jnp.bfloat16)

    nrb = S // rb
    return pl.pallas_call(
        functools.partial(_conv_stats_kernel, rb=rb, n_s=S, co=co),
        grid=(B, nrb),
        in_specs=[
            pl.BlockSpec((1, (S + 2) * S, 3 * ic), lambda b, r: (b, 0, 0)),
            pl.BlockSpec((3, 3 * ic, co), lambda b, r: (0, 0, 0)),
        ],
        out_specs=[
            pl.BlockSpec((1, rb * S, co), lambda b, r: (b, r, 0)),
            pl.BlockSpec((2, co), lambda b, r: (0, 0)),
        ],
        out_shape=[
            jax.ShapeDtypeStruct((B, S * S, co), jnp.bfloat16),
            jax.ShapeDtypeStruct((2, co), jnp.float32),
        ],
    )(xf, w1t)


def _pass2(h1, stats, p, n_total, mb):
    B, ss, co = h1.shape
    no = p['w2'].shape[0]
    nop = ((no + 127) // 128) * 128

    M = B * ss
    h1f = h1.reshape(M, co)
    gb = jnp.stack([p['g'], p['b']], axis=0).astype(jnp.float32)
    w2t = jnp.pad(p['w2'].reshape(no, co).T, ((0, 0), (0, nop - no)))
    w2t = w2t.astype(jnp.bfloat16)
    b2p = jnp.pad(p['b2'], (0, nop - no)).reshape(1, nop).astype(jnp.float32)

    out = pl.pallas_call(
        functools.partial(_bn_proj_kernel, n_total=float(n_total), eps=1e-5),
        grid=(M // mb,),
        in_specs=[
            pl.BlockSpec((mb, co), lambda i: (i, 0)),
            pl.BlockSpec((2, co), lambda i: (0, 0)),
            pl.BlockSpec((2, co), lambda i: (0, 0)),
            pl.BlockSpec((co, nop), lambda i: (0, 0)),
            pl.BlockSpec((1, nop), lambda i: (0, 0)),
        ],
        out_specs=pl.BlockSpec((mb, nop), lambda i: (i, 0)),
        out_shape=jax.ShapeDtypeStruct((M, nop), jnp.float32),
    )(h1f, stats, gb, w2t, b2p)
    return out


_SCALE_CFG = ((32, 2048), (32, 2048), (16, 1024))


def kernel(feat0, feat1, feat2, params):
    outs = []
    for x, p, (rb, mb) in zip((feat0, feat1, feat2), params, _SCALE_CFG):
        B, _, S, _ = x.shape
        no = p['w2'].shape[0]
        h1, stats = _pass1(x, p, rb)
        out = _pass2(h1, stats, p, n_total=B * S * S, mb=mb)
        nop = out.shape[-1]
        outs.append(out.reshape(B, S, S, nop)[..., :no])
    return tuple(outs)
